# baseline (speedup 1.0000x reference)
import jax
import jax.numpy as jnp
from jax.experimental import pallas as pl
from jax.experimental.pallas import tpu as pltpu


def _half(zbt, w1_ref, w2_ref, b1_ref, b2_ref):
    e, tc = zbt.shape
    nw = tc // 512
    zz = zbt.astype(jnp.bfloat16).reshape(e, nw, 512).transpose(1, 0, 2).reshape(nw, 512 * e)
    c1 = jnp.dot(zz, w1_ref[...], preferred_element_type=jnp.float32) + b1_ref[...]
    c2 = jnp.dot(zz, w2_ref[...], preferred_element_type=jnp.float32) + b2_ref[...]
    g = c1 * jax.nn.sigmoid(c2)
    return jnp.max(g, axis=0, keepdims=True)


def _malconv_kernel(za_ref, zb_ref, w1_ref, w2_ref, b1_ref, b2_ref, out_ref):
    ga = _half(za_ref[0], w1_ref, w2_ref, b1_ref, b2_ref)
    gb = _half(zb_ref[0], w1_ref, w2_ref, b1_ref, b2_ref)
    out_ref[0] = jnp.maximum(ga, gb)


def kernel(z, W1, b1, W2, b2):
    B, T, E = z.shape
    C, _, K = W1.shape
    KE = K * E
    zt = jnp.transpose(z, (0, 2, 1))
    W1t = W1.transpose(1, 2, 0).reshape(KE, C).astype(jnp.bfloat16)
    W2t = W2.transpose(1, 2, 0).reshape(KE, C).astype(jnp.bfloat16)
    out = pl.pallas_call(
        _malconv_kernel,
        grid=(B,),
        in_specs=[
            pl.BlockSpec((1, E, T // 2), lambda b: (b, 0, 0)),
            pl.BlockSpec((1, E, T // 2), lambda b: (b, 0, 1)),
            pl.BlockSpec((KE, C), lambda b: (0, 0)),
            pl.BlockSpec((KE, C), lambda b: (0, 0)),
            pl.BlockSpec((1, C), lambda b: (0, 0)),
            pl.BlockSpec((1, C), lambda b: (0, 0)),
        ],
        out_specs=pl.BlockSpec((1, 1, C), lambda b: (b, 0, 0)),
        out_shape=jax.ShapeDtypeStruct((B, 1, C), jnp.float32),
        compiler_params=pltpu.CompilerParams(
            dimension_semantics=("parallel",),
        ),
    )(zt, zt, W1t, W2t, b1.reshape(1, C), b2.reshape(1, C))
    return out.reshape(B, C)


# speedup vs baseline: 1.0743x; 1.0743x over previous
import jax
import jax.numpy as jnp
from jax.experimental import pallas as pl
from jax.experimental.pallas import tpu as pltpu


def _malconv_kernel(zt_ref, w1_ref, w2_ref, b1_ref, b2_ref, out_ref, w1s, w2s):
    zbt = zt_ref[0]
    e, tc = zbt.shape
    nw = tc // 512

    @pl.when(pl.program_id(0) == 0)
    def _():
        for ei in range(e):
            w1s[ei * 512:(ei + 1) * 512, :] = w1_ref[:, ei, :].T.astype(jnp.bfloat16)
            w2s[ei * 512:(ei + 1) * 512, :] = w2_ref[:, ei, :].T.astype(jnp.bfloat16)

    zz = zbt.astype(jnp.bfloat16).reshape(e, nw, 512).transpose(1, 0, 2).reshape(nw, 512 * e)
    c1 = jnp.dot(zz, w1s[...], preferred_element_type=jnp.float32) + b1_ref[...]
    c2 = jnp.dot(zz, w2s[...], preferred_element_type=jnp.float32) + b2_ref[...]
    g = c1 * jax.nn.sigmoid(c2)
    out_ref[0] = jnp.max(g, axis=0, keepdims=True)


def kernel(z, W1, b1, W2, b2):
    B, T, E = z.shape
    C, _, K = W1.shape
    KE = K * E
    zt = jnp.transpose(z, (0, 2, 1))
    out = pl.pallas_call(
        _malconv_kernel,
        grid=(B,),
        in_specs=[
            pl.BlockSpec((1, E, T), lambda b: (b, 0, 0)),
            pl.BlockSpec((C, E, K), lambda b: (0, 0, 0)),
            pl.BlockSpec((C, E, K), lambda b: (0, 0, 0)),
            pl.BlockSpec((1, C), lambda b: (0, 0)),
            pl.BlockSpec((1, C), lambda b: (0, 0)),
        ],
        out_specs=pl.BlockSpec((1, 1, C), lambda b: (b, 0, 0)),
        out_shape=jax.ShapeDtypeStruct((B, 1, C), jnp.float32),
        scratch_shapes=[
            pltpu.VMEM((KE, C), jnp.bfloat16),
            pltpu.VMEM((KE, C), jnp.bfloat16),
        ],
        compiler_params=pltpu.CompilerParams(
            dimension_semantics=("arbitrary",),
        ),
    )(zt, W1, W2, b1.reshape(1, C), b2.reshape(1, C))
    return out.reshape(B, C)
